# baseline (speedup 1.0000x reference)
import functools

import jax
import jax.numpy as jnp
from jax import lax
from jax.experimental import pallas as pl
from jax.experimental.pallas import tpu as pltpu
from jax.experimental.pallas import tpu_sc as plsc

DIM = 32
_info = plsc.get_sparse_core_info()
NC, NS = _info.num_cores, _info.num_subcores
NW = NC * NS

SEQ = 200
BATCH = 4096
B_TOTAL = BATCH * SEQ
B_PER_W = B_TOTAL // NW
CHUNK = 512
N_CHUNKS = B_PER_W // CHUNK
BLK_PER_T = BATCH // CHUNK
BJ_PER_CHUNK = CHUNK // 128


@functools.partial(
    pl.kernel,
    mesh=plsc.VectorSubcoreMesh(core_axis_name="c", subcore_axis_name="s"),
    compiler_params=pltpu.CompilerParams(
        use_tc_tiling_on_sc=False, needs_layout_passes=False),
    out_type=jax.ShapeDtypeStruct((SEQ, 4, DIM, 8, 128), jnp.float32),
    scratch_types=[
        pltpu.VMEM((2, CHUNK), jnp.int32),
        pltpu.VMEM((2, CHUNK, DIM), jnp.float32),
        pltpu.VMEM((2, 4, BJ_PER_CHUNK, 8, 128), jnp.float32),
    ] + [pltpu.SemaphoreType.DMA] * 6,
)
def _emb_fused(idx_hbm, table_hbm, out_hbm, idx_v, rows_v, tr_v, *sems):
    sem_g = sems[:2]
    sem_w = sems[2:4]
    sem_x = sems[4:]
    wid = lax.axis_index("s") * NC + lax.axis_index("c")
    blk0 = wid * N_CHUNKS

    lane = lax.iota(jnp.int32, 16)
    col_idx = [jnp.full((16,), d, jnp.int32) for d in range(DIM)]

    def issue_gather(i, p):
        off = (blk0 + i) * CHUNK
        pltpu.sync_copy(idx_hbm.at[pl.ds(off, CHUNK)], idx_v.at[p])
        pltpu.async_copy(table_hbm.at[idx_v.at[p]], rows_v.at[p], sem_g[p])

    def issue_idx(i, p):
        off = (blk0 + i) * CHUNK
        pltpu.async_copy(idx_hbm.at[pl.ds(off, CHUNK)], idx_v.at[p], sem_x[p])

    def wait_idx_issue_gather(i, p):
        off = (blk0 + i) * CHUNK
        pltpu.make_async_copy(
            idx_hbm.at[pl.ds(off, CHUNK)], idx_v.at[p], sem_x[p]).wait()
        pltpu.async_copy(table_hbm.at[idx_v.at[p]], rows_v.at[p], sem_g[p])

    def wait_gather(p):
        pltpu.make_async_copy(
            table_hbm.at[idx_v.at[p]], rows_v.at[p], sem_g[p]).wait()

    def out_slices(i, p):
        blk = blk0 + i
        t = blk // BLK_PER_T
        bj0 = (blk % BLK_PER_T) * BJ_PER_CHUNK
        return [(tr_v.at[p, di], out_hbm.at[t, di, pl.ds(bj0, BJ_PER_CHUNK)])
                for di in range(4)]

    def issue_out(i, p):
        for src, dst in out_slices(i, p):
            pltpu.async_copy(src, dst, sem_w[p])

    def wait_out(i, p):
        for src, dst in out_slices(i, p):
            pltpu.make_async_copy(src, dst, sem_w[p]).wait()

    def transpose_block(p):
        def tbody(g, carry):
            row = lane + 16 * g
            bj = g // 8
            b16 = 16 * (g % 8)
            def store(d, v):
                tr_v[p, d // 8, bj, d % 8, pl.ds(b16, 16)] = v
            vals = [None] * DIM
            for d in range(DIM):
                vals[d] = plsc.load_gather(rows_v.at[p], [row, col_idx[d]])
                if d >= 16:
                    store(d - 16, vals[d - 16])
            for d in range(DIM - 16, DIM):
                store(d, vals[d])
            return carry
        lax.fori_loop(0, CHUNK // 16, tbody, 0)

    def step(i, p, first, last):
        wait_gather(p)
        if not last:
            issue_idx(i + 2, p)
        if not first:
            wait_out(i - 2, p)
        transpose_block(p)
        issue_out(i, p)
        if not last:
            wait_idx_issue_gather(i + 2, p)

    issue_gather(0, 0)
    issue_gather(1, 1)
    for p in range(2):
        step(p, p, True, False)

    def body(g, carry):
        for p in range(2):
            step(2 * g + p, p, False, False)
        return carry

    lax.fori_loop(1, N_CHUNKS // 2 - 1, body, 0)

    for p in range(2):
        step(N_CHUNKS - 2 + p, p, False, True)
    for p in range(2):
        wait_out(N_CHUNKS - 2 + p, p)


def kernel(wordBatch, table):
    flat_idx = wordBatch.T.reshape(-1)
    out5 = _emb_fused(flat_idx, table)
    out = out5.transpose(0, 1, 3, 2, 4).reshape(SEQ, DIM, BATCH)
    return out.transpose(2, 0, 1)


# speedup vs baseline: 1.0062x; 1.0062x over previous
import functools

import jax
import jax.numpy as jnp
from jax import lax
from jax.experimental import pallas as pl
from jax.experimental.pallas import tpu as pltpu
from jax.experimental.pallas import tpu_sc as plsc

DIM = 32
_info = plsc.get_sparse_core_info()
NC, NS = _info.num_cores, _info.num_subcores
NW = NC * NS

SEQ = 200
BATCH = 4096
B_TOTAL = BATCH * SEQ
B_PER_W = B_TOTAL // NW
CHUNK = 512
N_CHUNKS = B_PER_W // CHUNK
BLK_PER_T = BATCH // CHUNK
BJ_PER_CHUNK = CHUNK // 128


@functools.partial(
    pl.kernel,
    mesh=plsc.VectorSubcoreMesh(core_axis_name="c", subcore_axis_name="s"),
    compiler_params=pltpu.CompilerParams(
        use_tc_tiling_on_sc=False, needs_layout_passes=False),
    out_type=jax.ShapeDtypeStruct((SEQ, 4, DIM, 8, 128), jnp.float32),
    scratch_types=[
        pltpu.VMEM((2, CHUNK), jnp.int32),
        pltpu.VMEM((2, CHUNK, DIM), jnp.float32),
        pltpu.VMEM((2, 4, BJ_PER_CHUNK, 8, 128), jnp.float32),
    ] + [pltpu.SemaphoreType.DMA] * 6,
)
def _emb_fused(idx_hbm, table_hbm, out_hbm, idx_v, rows_v, tr_v, *sems):
    sem_g = sems[:2]
    sem_w = sems[2:4]
    sem_x = sems[4:]
    wid = lax.axis_index("s") * NC + lax.axis_index("c")
    blk0 = wid * N_CHUNKS

    lane = lax.iota(jnp.int32, 16)
    col_idx = [jnp.full((16,), d, jnp.int32) for d in range(DIM)]

    def issue_gather(i, p):
        off = (blk0 + i) * CHUNK
        pltpu.sync_copy(idx_hbm.at[pl.ds(off, CHUNK)], idx_v.at[p])
        pltpu.async_copy(table_hbm.at[idx_v.at[p]], rows_v.at[p], sem_g[p])

    def issue_idx(i, p):
        off = (blk0 + i) * CHUNK
        pltpu.async_copy(idx_hbm.at[pl.ds(off, CHUNK)], idx_v.at[p], sem_x[p])

    def wait_idx_issue_gather(i, p):
        off = (blk0 + i) * CHUNK
        pltpu.make_async_copy(
            idx_hbm.at[pl.ds(off, CHUNK)], idx_v.at[p], sem_x[p]).wait()
        pltpu.async_copy(table_hbm.at[idx_v.at[p]], rows_v.at[p], sem_g[p])

    def wait_gather(p):
        pltpu.make_async_copy(
            table_hbm.at[idx_v.at[p]], rows_v.at[p], sem_g[p]).wait()

    def out_slices(i, p):
        blk = blk0 + i
        t = blk // BLK_PER_T
        bj0 = (blk % BLK_PER_T) * BJ_PER_CHUNK
        return [(tr_v.at[p, di], out_hbm.at[t, di, pl.ds(bj0, BJ_PER_CHUNK)])
                for di in range(4)]

    def issue_out(i, p):
        for src, dst in out_slices(i, p):
            pltpu.async_copy(src, dst, sem_w[p])

    def wait_out(i, p):
        for src, dst in out_slices(i, p):
            pltpu.make_async_copy(src, dst, sem_w[p]).wait()

    def transpose_block(p):
        def tbody(g, carry):
            row = lane + 16 * g
            bj = g // 8
            b16 = 16 * (g % 8)
            def store(d, v):
                tr_v[p, d // 8, bj, d % 8, pl.ds(b16, 16)] = v
            vals = [None] * DIM
            for d in range(DIM):
                vals[d] = plsc.load_gather(rows_v.at[p], [row, col_idx[d]])
                if d >= 8:
                    store(d - 8, vals[d - 8])
            for d in range(DIM - 8, DIM):
                store(d, vals[d])
            return carry
        lax.fori_loop(0, CHUNK // 16, tbody, 0)

    def step(i, p, first, last):
        wait_gather(p)
        if not last:
            issue_idx(i + 2, p)
        if not first:
            wait_out(i - 2, p)
        transpose_block(p)
        issue_out(i, p)
        if not last:
            wait_idx_issue_gather(i + 2, p)

    issue_gather(0, 0)
    issue_gather(1, 1)
    for p in range(2):
        step(p, p, True, False)

    def body(g, carry):
        for p in range(2):
            step(2 * g + p, p, False, False)
        return carry

    lax.fori_loop(1, N_CHUNKS // 2 - 1, body, 0)

    for p in range(2):
        step(N_CHUNKS - 2 + p, p, False, True)
    for p in range(2):
        wait_out(N_CHUNKS - 2 + p, p)


def kernel(wordBatch, table):
    flat_idx = wordBatch.T.reshape(-1)
    out5 = _emb_fused(flat_idx, table)
    out = out5.transpose(0, 1, 3, 2, 4).reshape(SEQ, DIM, BATCH)
    return out.transpose(2, 0, 1)
